# baseline (speedup 1.0000x reference)
import jax
import jax.numpy as jnp
from jax.experimental import pallas as pl
from jax.experimental.pallas import tpu as pltpu

_N, _DEG, _DIN, _DOUT = 10000, 32, 128, 128
_BLOCK = 800


def _fused_body(x_ref, nb_ref, wxt_ref, wnt_ref, ox_ref, onb_ref):
    nb = nb_ref[...]
    wxt = wxt_ref[...]
    onb = jnp.dot(nb.reshape(_BLOCK * _DEG, _DIN), wxt,
                  preferred_element_type=jnp.float32)
    onb_ref[...] = onb.reshape(_BLOCK, _DEG, _DOUT)
    f = jnp.sum(nb, axis=1) * (1.0 / _DEG)
    ox_ref[...] = (
        jnp.dot(x_ref[...], wxt, preferred_element_type=jnp.float32)
        + jnp.dot(f, wnt_ref[...], preferred_element_type=jnp.float32)
    )


def kernel(x, neighbor, Wx, Wn):
    wxt = Wx.T
    wnt = Wn.T
    out_x, out_nb = pl.pallas_call(
        _fused_body,
        grid=(pl.cdiv(_N, _BLOCK),),
        in_specs=[
            pl.BlockSpec((_BLOCK, _DIN), lambda i: (i, 0)),
            pl.BlockSpec((_BLOCK, _DEG, _DIN), lambda i: (i, 0, 0)),
            pl.BlockSpec((_DIN, _DOUT), lambda i: (0, 0)),
            pl.BlockSpec((_DIN, _DOUT), lambda i: (0, 0)),
        ],
        out_specs=[
            pl.BlockSpec((_BLOCK, _DOUT), lambda i: (i, 0)),
            pl.BlockSpec((_BLOCK, _DEG, _DOUT), lambda i: (i, 0, 0)),
        ],
        out_shape=[
            jax.ShapeDtypeStruct((_N, _DOUT), jnp.float32),
            jax.ShapeDtypeStruct((_N, _DEG, _DOUT), jnp.float32),
        ],
        compiler_params=pltpu.CompilerParams(
            dimension_semantics=("parallel",)),
    )(x, neighbor, wxt, wnt)
    return (out_x, out_nb)
